# baseline (speedup 1.0000x reference)
import functools

import jax
import jax.numpy as jnp
from jax.experimental import pallas as pl
from jax.experimental.pallas import tpu as pltpu

_TOP_K = 0.1
_UNIFORM_R = 0.5


def _encode_body(f_ref, w_ref, b_ref, enc_ref, comp_ref):
    x = f_ref[0]
    e = jnp.dot(w_ref[...], x, preferred_element_type=jnp.float32) + b_ref[...]
    enc_ref[0] = e
    comp_ref[0, 0] = jnp.sum(e, axis=0)


def _encode_compress(features_flat, W_conv, b_col, chunk):
    N, C, HW = features_flat.shape
    grid = (N, HW // chunk)
    return pl.pallas_call(
        _encode_body,
        grid=grid,
        in_specs=[
            pl.BlockSpec((1, C, chunk), lambda n, c: (n, 0, c)),
            pl.BlockSpec((C, C), lambda n, c: (0, 0)),
            pl.BlockSpec((C, 1), lambda n, c: (0, 0)),
        ],
        out_specs=[
            pl.BlockSpec((1, C, chunk), lambda n, c: (n, 0, c)),
            pl.BlockSpec((1, 1, chunk), lambda n, c: (n, 0, c)),
        ],
        out_shape=[
            jax.ShapeDtypeStruct((N, C, HW), jnp.float32),
            jax.ShapeDtypeStruct((N, 1, HW), jnp.float32),
        ],
    )(features_flat, W_conv, b_col)


def kernel(features, W_conv, b_conv):
    N, C, H, W = features.shape
    HW = H * W
    feats = features.reshape(N, C, HW)
    encoded_flat, compressed = _encode_compress(feats, W_conv, b_conv.reshape(C, 1), 2048)
    compressed = compressed.reshape(N, HW)
    encoded = encoded_flat.reshape(N, C, H, W)

    k_full = int(HW * _TOP_K)
    k = int(k_full * _UNIFORM_R)
    _, indices = jax.lax.top_k(compressed, k)
    perm = jax.random.permutation(jax.random.key(42), N)
    indices = indices[perm]
    h = indices // W
    w = indices % W
    sparse_indices = jnp.stack([h, w], axis=-1).astype(jnp.int32)
    sparse_features = jnp.take_along_axis(compressed, indices, axis=1)
    return sparse_features, sparse_indices, encoded
